# Initial kernel scaffold; baseline (speedup 1.0000x reference)
import jax
import jax.numpy as jnp
from jax.experimental import pallas as pl


def kernel(x, edge_index, W1, b1, gamma, beta, W2, b2):
    raise NotImplementedError("write your pallas kernel here")


# speedup vs baseline: 10.0820x; 10.0820x over previous
import functools

import jax
import jax.numpy as jnp
from jax import lax
from jax.experimental import pallas as pl
from jax.experimental.pallas import tpu as pltpu
from jax.experimental.pallas import tpu_sc as plsc

N = 10000
E = 320000
D = 128
H = 64

NC = 2
NS = 16
NT = NC * NS
C = 80
CH = E // (NT * C)
N2 = 10240
RPT = N2 // NS


def _make_prop(F, gather):
    mesh = plsc.VectorSubcoreMesh(core_axis_name="c", subcore_axis_name="s")
    scratch = []
    if gather:
        scratch.append(pltpu.VMEM((CH, C), jnp.int32))
    scratch += [
        pltpu.VMEM((CH, C), jnp.int32),
        pltpu.VMEM((C, F), jnp.float32),
        pltpu.VMEM_SHARED((N2, F), jnp.float32),
        pltpu.SemaphoreType.DMA,
    ]

    def body_gather(u, srcg, dstg, z, out, srcbuf, dstbuf, rowbuf, acc, sem):
        c = lax.axis_index("c")
        s = lax.axis_index("s")
        wid = s * NC + c
        pltpu.sync_copy(z.at[pl.ds(s * RPT, RPT)], acc.at[pl.ds(s * RPT, RPT)])
        pltpu.sync_copy(srcg.at[wid], srcbuf)
        pltpu.sync_copy(dstg.at[wid], dstbuf)
        plsc.subcore_barrier()

        def step(j, carry):
            pltpu.async_copy(u.at[srcbuf.at[j]], rowbuf, sem).wait()
            pltpu.sync_copy(rowbuf, acc.at[dstbuf.at[j]], add=True)
            return carry

        lax.fori_loop(0, CH, step, 0)
        plsc.subcore_barrier()
        pltpu.sync_copy(acc.at[pl.ds(s * RPT, RPT)],
                        out.at[c, pl.ds(s * RPT, RPT)])

    def body_ones(dstg, z, out, dstbuf, rowbuf, acc, sem):
        c = lax.axis_index("c")
        s = lax.axis_index("s")
        wid = s * NC + c
        pltpu.sync_copy(z.at[pl.ds(s * RPT, RPT)], acc.at[pl.ds(s * RPT, RPT)])
        pltpu.sync_copy(dstg.at[wid], dstbuf)

        def fill(i, carry):
            rowbuf[i // 8, pl.ds((i % 8) * 16, 16)] = jnp.full(
                (16,), 1.0, jnp.float32)
            return carry

        lax.fori_loop(0, C * 8, fill, 0)
        plsc.subcore_barrier()

        def step(j, carry):
            pltpu.sync_copy(rowbuf, acc.at[dstbuf.at[j]], add=True)
            return carry

        lax.fori_loop(0, CH, step, 0)
        plsc.subcore_barrier()
        pltpu.sync_copy(acc.at[pl.ds(s * RPT, RPT)],
                        out.at[c, pl.ds(s * RPT, RPT)])

    body = body_gather if gather else body_ones
    return pl.kernel(
        body,
        out_type=jax.ShapeDtypeStruct((NC, N2, F), jnp.float32),
        mesh=mesh,
        scratch_types=scratch,
    )


_deg_sc = _make_prop(128, gather=False)
_prop128 = _make_prop(128, gather=True)


def _prep_body(degp_ref, x_ref, dinv_ref, u1_ref):
    deg = (degp_ref[0] + degp_ref[1])[:N, 0:1]
    dinv = jnp.where(deg > 0, lax.rsqrt(deg), 0.0)
    dinv_ref[...] = dinv
    u1_ref[...] = x_ref[...] * dinv


def _scale_body(gp_ref, dinv_ref, p_ref, u_ref):
    g = (gp_ref[0] + gp_ref[1])[:N]
    dv = dinv_ref[...]
    p = g * dv
    p_ref[...] = p
    u_ref[...] = p * dv


def _conv1_body(x_ref, p1_ref, gp2_ref, dinv_ref, w_ref, b_ref,
                h_ref, stats_ref):
    p2 = (gp2_ref[0] + gp2_ref[1])[:N] * dinv_ref[...]
    f32 = jnp.float32
    h = jnp.dot(x_ref[...], w_ref[0] - w_ref[2], preferred_element_type=f32)
    h = h - jnp.dot(p1_ref[...], w_ref[1], preferred_element_type=f32)
    h = h + 2.0 * jnp.dot(p2, w_ref[2], preferred_element_type=f32)
    h = h + b_ref[...]
    h_ref[...] = h
    stats_ref[0, :] = jnp.sum(h, axis=0)
    stats_ref[1, :] = jnp.sum(h * h, axis=0)


def _bn_body(hp_ref, stats_ref, gamma_ref, beta_ref, dinv_ref, h_ref, u_ref):
    inv_n = 1.0 / N
    mu = stats_ref[0, :] * inv_n
    var = stats_ref[1, :] * inv_n - mu * mu
    hn = (hp_ref[...] - mu) / jnp.sqrt(var + 1e-5) * gamma_ref[...] \
        + beta_ref[...]
    h = jnp.where(hn >= 0, hn, 0.01 * hn)
    h_ref[...] = h
    u_ref[...] = jnp.concatenate(
        [h * dinv_ref[...], jnp.zeros((N, D - H), jnp.float32)], axis=1)


def _conv2_body(h_ref, p3_ref, gp4_ref, dinv_ref, w_ref, b_ref, x_ref,
                xh_ref, loss_ref):
    p4 = (gp4_ref[0] + gp4_ref[1])[:N, :H] * dinv_ref[...]
    f32 = jnp.float32
    xh = jnp.dot(h_ref[...], w_ref[0] - w_ref[2], preferred_element_type=f32)
    xh = xh - jnp.dot(p3_ref[...][:, :H], w_ref[1], preferred_element_type=f32)
    xh = xh + 2.0 * jnp.dot(p4, w_ref[2], preferred_element_type=f32)
    xh = xh + b_ref[...]
    xh_ref[...] = xh
    diff = xh - x_ref[...]
    loss_ref[...] = (jnp.sum(diff * diff) * (1.0 / (N * D))).reshape(1, 1)


def _sds(shape):
    return jax.ShapeDtypeStruct(shape, jnp.float32)


def kernel(x, edge_index, W1, b1, gamma, beta, W2, b2):
    src = edge_index[0].reshape(NT, CH, C)
    dst = edge_index[1].reshape(NT, CH, C)
    z128 = jnp.zeros((N2, 128), jnp.float32)

    degp = _deg_sc(dst, z128)
    dinv, u1 = pl.pallas_call(
        _prep_body, out_shape=(_sds((N, 1)), _sds((N, D))))(degp, x)
    g1p = _prop128(u1, src, dst, z128)
    p1, u2 = pl.pallas_call(
        _scale_body, out_shape=(_sds((N, D)), _sds((N, D))))(g1p, dinv)
    g2p = _prop128(u2, src, dst, z128)
    h_pre, stats = pl.pallas_call(
        _conv1_body, out_shape=(_sds((N, H)), _sds((2, H))))(
            x, p1, g2p, dinv, W1, b1)
    h, u3 = pl.pallas_call(
        _bn_body, out_shape=(_sds((N, H)), _sds((N, D))))(
            h_pre, stats, gamma, beta, dinv)
    g3p = _prop128(u3, src, dst, z128)
    p3, u4 = pl.pallas_call(
        _scale_body, out_shape=(_sds((N, D)), _sds((N, D))))(g3p, dinv)
    g4p = _prop128(u4, src, dst, z128)
    x_hat, loss2 = pl.pallas_call(
        _conv2_body, out_shape=(_sds((N, D)), _sds((1, 1))))(
            h, p3, g4p, dinv, W2, b2, x)
    return (x_hat, loss2[0, 0])


# speedup vs baseline: 16.0689x; 1.5938x over previous
import jax
import jax.numpy as jnp
from jax import lax
from jax.experimental import pallas as pl
from jax.experimental.pallas import tpu as pltpu
from jax.experimental.pallas import tpu_sc as plsc

N = 10000
E = 320000
D = 128
H = 64

NC = 2
NS = 16
NT = NC * NS
C = 128
CH = 80
SB = 40
NSB = CH // SB
EPT = CH * C
PE = NT * EPT
NPAIR = CH // 2
N2 = 10240
RPT = N2 // NS


def _make_prop(F, gather):
    mesh = plsc.VectorSubcoreMesh(core_axis_name="c", subcore_axis_name="s")
    if gather:
        scratch = [
            pltpu.VMEM((SB, C), jnp.int32),
            pltpu.VMEM((SB, C), jnp.int32),
            pltpu.VMEM((C, F), jnp.float32),
            pltpu.VMEM((C, F), jnp.float32),
            pltpu.VMEM_SHARED((N2, F), jnp.float32),
            pltpu.SemaphoreType.DMA,
            pltpu.SemaphoreType.DMA,
        ]
    else:
        scratch = [
            pltpu.VMEM((SB, C), jnp.int32),
            pltpu.VMEM((C, F), jnp.float32),
            pltpu.VMEM_SHARED((N2, F), jnp.float32),
        ]

    def body_gather(u, srcg, dstg, z, out, srcbuf, dstbuf, rb0, rb1, acc,
                    gs0, gs1):
        c = lax.axis_index("c")
        s = lax.axis_index("s")
        wid = s * NC + c
        pltpu.sync_copy(z.at[pl.ds(s * RPT, RPT)], acc.at[pl.ds(s * RPT, RPT)])
        plsc.subcore_barrier()

        def step(i, carry):
            j = 2 * i
            pltpu.async_copy(u.at[srcbuf.at[j + 1]], rb1, gs1)
            pltpu.make_async_copy(u.at[srcbuf.at[j]], rb0, gs0).wait()
            pltpu.sync_copy(rb0, acc.at[dstbuf.at[j]], add=True)
            jn = jnp.minimum(j + 2, SB - 1)
            pltpu.async_copy(u.at[srcbuf.at[jn]], rb0, gs0)
            pltpu.make_async_copy(u.at[srcbuf.at[j + 1]], rb1, gs1).wait()
            pltpu.sync_copy(rb1, acc.at[dstbuf.at[j + 1]], add=True)
            return carry

        for sb in range(NSB):
            pltpu.sync_copy(srcg.at[wid, pl.ds(sb * SB, SB)], srcbuf)
            pltpu.sync_copy(dstg.at[wid, pl.ds(sb * SB, SB)], dstbuf)
            pltpu.async_copy(u.at[srcbuf.at[0]], rb0, gs0)
            lax.fori_loop(0, SB // 2, step, 0)
            pltpu.make_async_copy(u.at[srcbuf.at[0]], rb0, gs0).wait()
        plsc.subcore_barrier()
        pltpu.sync_copy(acc.at[pl.ds(s * RPT, RPT)],
                        out.at[c, pl.ds(s * RPT, RPT)])

    def body_ones(dstg, z, out, dstbuf, rowbuf, acc):
        c = lax.axis_index("c")
        s = lax.axis_index("s")
        wid = s * NC + c
        pltpu.sync_copy(z.at[pl.ds(s * RPT, RPT)], acc.at[pl.ds(s * RPT, RPT)])

        def fill(i, carry):
            rowbuf[i // 8, pl.ds((i % 8) * 16, 16)] = jnp.full(
                (16,), 1.0, jnp.float32)
            return carry

        lax.fori_loop(0, C * 8, fill, 0)
        plsc.subcore_barrier()

        def step(j, carry):
            pltpu.sync_copy(rowbuf, acc.at[dstbuf.at[j]], add=True)
            return carry

        for sb in range(NSB):
            pltpu.sync_copy(dstg.at[wid, pl.ds(sb * SB, SB)], dstbuf)
            lax.fori_loop(0, SB, step, 0)
        plsc.subcore_barrier()
        pltpu.sync_copy(acc.at[pl.ds(s * RPT, RPT)],
                        out.at[c, pl.ds(s * RPT, RPT)])

    body = body_gather if gather else body_ones
    return pl.kernel(
        body,
        out_type=jax.ShapeDtypeStruct((NC, N2, F), jnp.float32),
        mesh=mesh,
        scratch_types=scratch,
    )


_deg_sc = _make_prop(128, gather=False)
_prop128 = _make_prop(128, gather=True)


def _prep_body(degp_ref, x_ref, dinv_ref, u1_ref):
    deg = (degp_ref[0] + degp_ref[1])[:N, 0:1]
    dinv = jnp.where(deg > 0, lax.rsqrt(deg), 0.0)
    dinv_ref[...] = dinv
    u1_ref[:N] = x_ref[...] * dinv
    u1_ref[N:] = jnp.zeros((N2 - N, D), jnp.float32)


def _scale_body(gp_ref, dinv_ref, p_ref, u_ref):
    g = (gp_ref[0] + gp_ref[1])[:N]
    dv = dinv_ref[...]
    p = g * dv
    p_ref[...] = p
    u_ref[:N] = p * dv
    u_ref[N:] = jnp.zeros((N2 - N, D), jnp.float32)


def _conv1_body(x_ref, p1_ref, gp2_ref, dinv_ref, w_ref, b_ref,
                h_ref, stats_ref):
    p2 = (gp2_ref[0] + gp2_ref[1])[:N] * dinv_ref[...]
    f32 = jnp.float32
    h = jnp.dot(x_ref[...], w_ref[0] - w_ref[2], preferred_element_type=f32)
    h = h - jnp.dot(p1_ref[...], w_ref[1], preferred_element_type=f32)
    h = h + 2.0 * jnp.dot(p2, w_ref[2], preferred_element_type=f32)
    h = h + b_ref[...]
    h_ref[...] = h
    stats_ref[0, :] = jnp.sum(h, axis=0)
    stats_ref[1, :] = jnp.sum(h * h, axis=0)


def _bn_body(hp_ref, stats_ref, gamma_ref, beta_ref, dinv_ref, h_ref, u_ref):
    inv_n = 1.0 / N
    mu = stats_ref[0, :] * inv_n
    var = stats_ref[1, :] * inv_n - mu * mu
    hn = (hp_ref[...] - mu) / jnp.sqrt(var + 1e-5) * gamma_ref[...] \
        + beta_ref[...]
    h = jnp.where(hn >= 0, hn, 0.01 * hn)
    h_ref[...] = h
    u_ref[:N] = jnp.concatenate(
        [h * dinv_ref[...], jnp.zeros((N, D - H), jnp.float32)], axis=1)
    u_ref[N:] = jnp.zeros((N2 - N, D), jnp.float32)


def _conv2_body(h_ref, p3_ref, gp4_ref, dinv_ref, w_ref, b_ref, x_ref,
                xh_ref, loss_ref):
    p4 = (gp4_ref[0] + gp4_ref[1])[:N, :H] * dinv_ref[...]
    f32 = jnp.float32
    xh = jnp.dot(h_ref[...], w_ref[0] - w_ref[2], preferred_element_type=f32)
    xh = xh - jnp.dot(p3_ref[...][:, :H], w_ref[1], preferred_element_type=f32)
    xh = xh + 2.0 * jnp.dot(p4, w_ref[2], preferred_element_type=f32)
    xh = xh + b_ref[...]
    xh_ref[...] = xh
    diff = xh - x_ref[...]
    loss_ref[...] = (jnp.sum(diff * diff) * (1.0 / (N * D))).reshape(1, 1)


def _sds(shape):
    return jax.ShapeDtypeStruct(shape, jnp.float32)


def kernel(x, edge_index, W1, b1, gamma, beta, W2, b2):
    pad = N + (jnp.arange(PE - E, dtype=jnp.int32) % (N2 - N))
    src = jnp.concatenate([edge_index[0], pad]).reshape(NT, CH, C)
    dst = jnp.concatenate([edge_index[1], pad]).reshape(NT, CH, C)
    z128 = jnp.zeros((N2, 128), jnp.float32)

    degp = _deg_sc(dst, z128)
    dinv, u1 = pl.pallas_call(
        _prep_body, out_shape=(_sds((N, 1)), _sds((N2, D))))(degp, x)
    g1p = _prop128(u1, src, dst, z128)
    p1, u2 = pl.pallas_call(
        _scale_body, out_shape=(_sds((N, D)), _sds((N2, D))))(g1p, dinv)
    g2p = _prop128(u2, src, dst, z128)
    h_pre, stats = pl.pallas_call(
        _conv1_body, out_shape=(_sds((N, H)), _sds((2, H))))(
            x, p1, g2p, dinv, W1, b1)
    h, u3 = pl.pallas_call(
        _bn_body, out_shape=(_sds((N, H)), _sds((N2, D))))(
            h_pre, stats, gamma, beta, dinv)
    g3p = _prop128(u3, src, dst, z128)
    p3, u4 = pl.pallas_call(
        _scale_body, out_shape=(_sds((N, D)), _sds((N2, D))))(g3p, dinv)
    g4p = _prop128(u4, src, dst, z128)
    x_hat, loss2 = pl.pallas_call(
        _conv2_body, out_shape=(_sds((N, D)), _sds((1, 1))))(
            h, p3, g4p, dinv, W2, b2, x)
    return (x_hat, loss2[0, 0])


# speedup vs baseline: 17.3331x; 1.0787x over previous
import jax
import jax.numpy as jnp
from jax import lax
from jax.experimental import pallas as pl
from jax.experimental.pallas import tpu as pltpu
from jax.experimental.pallas import tpu_sc as plsc

N = 10000
E = 320000
D = 128
H = 64

NC = 2
NS = 16
NT = NC * NS
C = 128
CH = 80
SB = 40
NSB = CH // SB
EPT = CH * C
PE = NT * EPT
NPAIR = CH // 2
N2 = 10240
RPT = N2 // NS


def _make_prop(F, gather):
    mesh = plsc.VectorSubcoreMesh(core_axis_name="c", subcore_axis_name="s")
    if gather:
        scratch = [
            pltpu.VMEM((SB, C), jnp.int32),
            pltpu.VMEM((SB, C), jnp.int32),
            pltpu.VMEM((C, F), jnp.float32),
            pltpu.VMEM((C, F), jnp.float32),
            pltpu.VMEM_SHARED((N2, F), jnp.float32),
            pltpu.SemaphoreType.DMA,
            pltpu.SemaphoreType.DMA,
        ]
    else:
        scratch = [
            pltpu.VMEM((SB, C), jnp.int32),
            pltpu.VMEM((C, F), jnp.float32),
            pltpu.VMEM_SHARED((N2, F), jnp.float32),
        ]

    def body_gather(u, srcg, dstg, z, out, srcbuf, dstbuf, rb0, rb1, acc,
                    gs0, gs1):
        c = lax.axis_index("c")
        s = lax.axis_index("s")
        wid = s * NC + c
        pltpu.sync_copy(z.at[pl.ds(s * RPT, RPT)], acc.at[pl.ds(s * RPT, RPT)])
        plsc.subcore_barrier()

        def step(i, carry):
            j = 2 * i
            pltpu.async_copy(u.at[srcbuf.at[j + 1]], rb1, gs1)
            pltpu.make_async_copy(u.at[srcbuf.at[j]], rb0, gs0).wait()
            pltpu.sync_copy(rb0, acc.at[dstbuf.at[j]], add=True)
            jn = jnp.minimum(j + 2, SB - 1)
            pltpu.async_copy(u.at[srcbuf.at[jn]], rb0, gs0)
            pltpu.make_async_copy(u.at[srcbuf.at[j + 1]], rb1, gs1).wait()
            pltpu.sync_copy(rb1, acc.at[dstbuf.at[j + 1]], add=True)
            return carry

        for sb in range(NSB):
            pltpu.sync_copy(srcg.at[wid, pl.ds(sb * SB, SB)], srcbuf)
            pltpu.sync_copy(dstg.at[wid, pl.ds(sb * SB, SB)], dstbuf)
            pltpu.async_copy(u.at[srcbuf.at[0]], rb0, gs0)
            lax.fori_loop(0, SB // 2, step, 0)
            pltpu.make_async_copy(u.at[srcbuf.at[0]], rb0, gs0).wait()
        plsc.subcore_barrier()
        pltpu.sync_copy(acc.at[pl.ds(s * RPT, RPT)],
                        out.at[c, pl.ds(s * RPT, RPT)])

    def body_ones(dstg, z, out, dstbuf, rowbuf, acc):
        c = lax.axis_index("c")
        s = lax.axis_index("s")
        wid = s * NC + c
        pltpu.sync_copy(z.at[pl.ds(s * RPT, RPT)], acc.at[pl.ds(s * RPT, RPT)])

        nf = F // 16

        def fill(i, carry):
            rowbuf[i // nf, pl.ds((i % nf) * 16, 16)] = jnp.full(
                (16,), 1.0, jnp.float32)
            return carry

        lax.fori_loop(0, C * nf, fill, 0)
        plsc.subcore_barrier()

        def step(j, carry):
            pltpu.sync_copy(rowbuf, acc.at[dstbuf.at[j]], add=True)
            return carry

        for sb in range(NSB):
            pltpu.sync_copy(dstg.at[wid, pl.ds(sb * SB, SB)], dstbuf)
            lax.fori_loop(0, SB, step, 0)
        plsc.subcore_barrier()
        pltpu.sync_copy(acc.at[pl.ds(s * RPT, RPT)],
                        out.at[c, pl.ds(s * RPT, RPT)])

    body = body_gather if gather else body_ones
    return pl.kernel(
        body,
        out_type=jax.ShapeDtypeStruct((NC, N2, F), jnp.float32),
        mesh=mesh,
        scratch_types=scratch,
    )


_deg_sc = _make_prop(32, gather=False)
_prop128 = _make_prop(128, gather=True)


def _prep_body(degp_ref, x_ref, dinv_ref, u1_ref):
    deg = (degp_ref[0] + degp_ref[1])[:N, 0:1]
    dinv = jnp.where(deg > 0, lax.rsqrt(deg), 0.0)
    dinv_ref[...] = dinv
    u1_ref[:N] = x_ref[...] * dinv
    u1_ref[N:] = jnp.zeros((N2 - N, D), jnp.float32)


def _scale_body(gp_ref, dinv_ref, p_ref, u_ref):
    g = (gp_ref[0] + gp_ref[1])[:N]
    dv = dinv_ref[...]
    p = g * dv
    p_ref[...] = p
    u_ref[:N] = p * dv
    u_ref[N:] = jnp.zeros((N2 - N, D), jnp.float32)


def _conv1bn_body(x_ref, p1_ref, gp2_ref, dinv_ref, w_ref, b_ref,
                  gamma_ref, beta_ref, h_ref, u_ref):
    p2 = (gp2_ref[0] + gp2_ref[1])[:N] * dinv_ref[...]
    f32 = jnp.float32
    hp = jnp.dot(x_ref[...], w_ref[0] - w_ref[2], preferred_element_type=f32)
    hp = hp - jnp.dot(p1_ref[...], w_ref[1], preferred_element_type=f32)
    hp = hp + 2.0 * jnp.dot(p2, w_ref[2], preferred_element_type=f32)
    hp = hp + b_ref[...]
    inv_n = 1.0 / N
    mu = jnp.sum(hp, axis=0) * inv_n
    var = jnp.sum(hp * hp, axis=0) * inv_n - mu * mu
    hn = (hp - mu) / jnp.sqrt(var + 1e-5) * gamma_ref[...] + beta_ref[...]
    h = jnp.where(hn >= 0, hn, 0.01 * hn)
    h_ref[...] = h
    u_ref[:N] = jnp.concatenate(
        [h * dinv_ref[...], jnp.zeros((N, D - H), jnp.float32)], axis=1)
    u_ref[N:] = jnp.zeros((N2 - N, D), jnp.float32)


def _conv2_body(h_ref, p3_ref, gp4_ref, dinv_ref, w_ref, b_ref, x_ref,
                xh_ref, loss_ref):
    p4 = (gp4_ref[0] + gp4_ref[1])[:N, :H] * dinv_ref[...]
    f32 = jnp.float32
    xh = jnp.dot(h_ref[...], w_ref[0] - w_ref[2], preferred_element_type=f32)
    xh = xh - jnp.dot(p3_ref[...][:, :H], w_ref[1], preferred_element_type=f32)
    xh = xh + 2.0 * jnp.dot(p4, w_ref[2], preferred_element_type=f32)
    xh = xh + b_ref[...]
    xh_ref[...] = xh
    diff = xh - x_ref[...]
    loss_ref[...] = (jnp.sum(diff * diff) * (1.0 / (N * D))).reshape(1, 1)


def _sds(shape):
    return jax.ShapeDtypeStruct(shape, jnp.float32)


def kernel(x, edge_index, W1, b1, gamma, beta, W2, b2):
    pad = N + (jnp.arange(PE - E, dtype=jnp.int32) % (N2 - N))
    src = jnp.concatenate([edge_index[0], pad]).reshape(NT, CH, C)
    dst = jnp.concatenate([edge_index[1], pad]).reshape(NT, CH, C)
    z128 = jnp.zeros((N2, 128), jnp.float32)

    z32 = jnp.zeros((N2, 32), jnp.float32)
    degp = _deg_sc(dst, z32)
    dinv, u1 = pl.pallas_call(
        _prep_body, out_shape=(_sds((N, 1)), _sds((N2, D))))(degp, x)
    g1p = _prop128(u1, src, dst, z128)
    p1, u2 = pl.pallas_call(
        _scale_body, out_shape=(_sds((N, D)), _sds((N2, D))))(g1p, dinv)
    g2p = _prop128(u2, src, dst, z128)
    h, u3 = pl.pallas_call(
        _conv1bn_body, out_shape=(_sds((N, H)), _sds((N2, D))))(
            x, p1, g2p, dinv, W1, b1, gamma, beta)
    g3p = _prop128(u3, src, dst, z128)
    p3, u4 = pl.pallas_call(
        _scale_body, out_shape=(_sds((N, D)), _sds((N2, D))))(g3p, dinv)
    g4p = _prop128(u4, src, dst, z128)
    x_hat, loss2 = pl.pallas_call(
        _conv2_body, out_shape=(_sds((N, D)), _sds((1, 1))))(
            h, p3, g4p, dinv, W2, b2, x)
    return (x_hat, loss2[0, 0])
